# baseline (speedup 1.0000x reference)
import jax
import jax.numpy as jnp
from jax.experimental import pallas as pl
from jax.experimental.pallas import tpu as pltpu

_B, _C, _H, _W = 8, 768, 56, 56
_G = 4
_g = _C // _G
_r = 16
_HID = _g // _r
_K = 7
_HK = _H // _K


def _body(x_ref, w1_ref, w2_ref, o_ref, att_ref):
    k = pl.program_id(1)

    @pl.when(k == 0)
    def _compute_att():
        xb = x_ref[0]
        pooled = jnp.sum(xb, axis=(1, 2), keepdims=False)
        pooled = pooled.reshape(1, _C) * (1.0 / (_H * _W))
        h = jax.nn.relu(
            jnp.dot(pooled, w1_ref[...], preferred_element_type=jnp.float32))
        att = jax.nn.sigmoid(
            jnp.dot(h, w2_ref[...], preferred_element_type=jnp.float32))
        att_ref[...] = att

    sl = pl.ds(k * _HK, _HK)
    o_ref[0] = x_ref[0, :, sl] * att_ref[0][:, None, None]


def kernel(x, W1, W2):
    W1f = jnp.zeros((_C, _G * _HID), jnp.float32)
    W2f = jnp.zeros((_G * _HID, _C), jnp.float32)
    for i in range(_G):
        W1f = W1f.at[i::_G, i * _HID:(i + 1) * _HID].set(W1[i])
        W2f = W2f.at[i * _HID:(i + 1) * _HID, i::_G].set(W2[i])

    return pl.pallas_call(
        _body,
        grid=(_B, _K),
        in_specs=[
            pl.BlockSpec((1, _C, _H, _W), lambda b, k: (b, 0, 0, 0)),
            pl.BlockSpec((_C, _G * _HID), lambda b, k: (0, 0)),
            pl.BlockSpec((_G * _HID, _C), lambda b, k: (0, 0)),
        ],
        out_specs=pl.BlockSpec((1, _C, _HK, _W), lambda b, k: (b, 0, k, 0)),
        out_shape=jax.ShapeDtypeStruct((_B, _C, _H, _W), jnp.float32),
        scratch_shapes=[pltpu.VMEM((1, _C), jnp.float32)],
        compiler_params=pltpu.CompilerParams(
            dimension_semantics=("arbitrary", "arbitrary"),
        ),
    )(x, W1f, W2f)


# speedup vs baseline: 10.0983x; 10.0983x over previous
import functools

import jax
import jax.numpy as jnp
from jax import lax
from jax.experimental import pallas as pl
from jax.experimental.pallas import tpu as pltpu
from jax.experimental.pallas import tpu_sc as plsc

_B, _C, _H, _W = 8, 768, 56, 56
_G = 4
_g = _C // _G
_HID = 12
_HW = _H * _W
_L = 16
_NV = _g // _L
_NS = _HW // _L
_CHUNK = 16
_NCHUNK = _g // _CHUNK


def _lane_total(scr, v):
    for k in (8, 4, 2, 1):
        scr[pl.ds(0, _L)] = v
        scr[pl.ds(_L, _L)] = v
        v = v + scr[pl.ds(k, _L)]
    return v


def _lane_splat(scr, tvec, v, j):
    onehot = jnp.where(tvec == j, 1.0, 0.0)
    return _lane_total(scr, v * onehot)


def _row_sum_vec(buf, j):
    zero = jnp.zeros((_L,), jnp.float32)

    def step(t, accs):
        a0, a1, a2, a3 = accs
        base = t * 4 * _L
        a0 = a0 + buf[j, pl.ds(base + 0 * _L, _L)]
        a1 = a1 + buf[j, pl.ds(base + 1 * _L, _L)]
        a2 = a2 + buf[j, pl.ds(base + 2 * _L, _L)]
        a3 = a3 + buf[j, pl.ds(base + 3 * _L, _L)]
        return (a0, a1, a2, a3)

    a0, a1, a2, a3 = lax.fori_loop(0, _NS // 4, step, (zero, zero, zero, zero))
    return (a0 + a1) + (a2 + a3)


def _sc_body(x_hbm, w1t_hbm, w2_hbm, out_hbm,
             buf0, buf1, idx0, idx1, pooled_v, att_v, w1_v, w2_v, rot_v,
             gsem0, gsem1, ssem0, ssem1, wsem):
    wid = lax.axis_index("s") * 2 + lax.axis_index("c")
    b = wid // _G
    i = wid % _G

    bufs = (buf0, buf1)
    idxs = (idx0, idx1)
    gsems = (gsem0, gsem1)
    ssems = (ssem0, ssem1)

    pltpu.async_copy(w1t_hbm.at[pl.ds(i, 1)], w1_v, wsem).wait()
    pltpu.async_copy(w2_hbm.at[pl.ds(i, 1)], w2_v, wsem).wait()

    row0 = b * _C + i
    tvec = lax.iota(jnp.int32, _L)

    def set_idx(s, c):
        idxs[s][...] = (row0 + 4 * _CHUNK * c) + 4 * tvec

    def start_gather(s, c):
        set_idx(s, c)
        pltpu.make_async_copy(x_hbm.at[idxs[s]], bufs[s], gsems[s]).start()

    def wait_gather(s):
        pltpu.make_async_copy(x_hbm.at[idxs[s]], bufs[s], gsems[s]).wait()

    start_gather(0, 0)
    for c in range(_NCHUNK):
        s = c % 2
        if c + 1 < _NCHUNK:
            start_gather(1 - s, c + 1)
        wait_gather(s)

        def red_row(j, sums_vec):
            sj = _lane_total(rot_v, _row_sum_vec(bufs[s], j))
            return jnp.where(tvec == j, sj, sums_vec)

        sums_vec = lax.fori_loop(0, _CHUNK, red_row,
                                 jnp.zeros((_L,), jnp.float32))
        pooled_v[pl.ds(c * _CHUNK, _CHUNK)] = sums_vec * (1.0 / _HW)

    hs = []
    for k in range(_HID):
        acc = jnp.zeros((_L,), jnp.float32)
        for m in range(_NV):
            acc = acc + (pooled_v[pl.ds(m * _L, _L)]
                         * w1_v[0, k, pl.ds(m * _L, _L)])
        hs.append(jnp.maximum(_lane_total(rot_v, acc), 0.0))
    for m in range(_NV):
        acc = jnp.zeros((_L,), jnp.float32)
        for k in range(_HID):
            acc = acc + hs[k] * w2_v[0, k, pl.ds(m * _L, _L)]
        att_v[pl.ds(m * _L, _L)] = 1.0 / (1.0 + jnp.exp(-acc))

    def start_scatter(s, c):
        pltpu.make_async_copy(bufs[s], out_hbm.at[idxs[s]], ssems[s]).start()

    def wait_scatter(s, c):
        pltpu.make_async_copy(bufs[s], out_hbm.at[idxs[s]], ssems[s]).wait()

    start_gather(0, 0)
    for c in range(_NCHUNK):
        s = c % 2
        wait_gather(s)
        if c + 1 < _NCHUNK:
            if c >= 1:
                wait_scatter(1 - s, c - 1)
            start_gather(1 - s, c + 1)

        av = att_v[pl.ds(c * _CHUNK, _CHUNK)]

        def scale_row(j, _):
            sj = _lane_splat(rot_v, tvec, av, j)

            def sstep(t, _2):
                base = t * _L
                sl = pl.ds(base, _L)
                bufs[s][j, sl] = bufs[s][j, sl] * sj
                return 0

            lax.fori_loop(0, _NS, sstep, 0)
            return 0

        lax.fori_loop(0, _CHUNK, scale_row, 0)
        start_scatter(s, c)
    wait_scatter(0, _NCHUNK - 2)
    wait_scatter(1, _NCHUNK - 1)


@functools.partial(jax.jit, static_argnums=())
def _sc_call(x2, w1t, w2):
    mesh = plsc.VectorSubcoreMesh(core_axis_name="c", subcore_axis_name="s")
    f = functools.partial(
        pl.kernel,
        out_type=jax.ShapeDtypeStruct((_B * _C, _HW), jnp.float32),
        mesh=mesh,
        compiler_params=pltpu.CompilerParams(use_tc_tiling_on_sc=False),
        scratch_types=[
            pltpu.VMEM((_CHUNK, _HW), jnp.float32),
            pltpu.VMEM((_CHUNK, _HW), jnp.float32),
            pltpu.VMEM((_L,), jnp.int32),
            pltpu.VMEM((_L,), jnp.int32),
            pltpu.VMEM((_g,), jnp.float32),
            pltpu.VMEM((_g,), jnp.float32),
            pltpu.VMEM((1, _HID, _g), jnp.float32),
            pltpu.VMEM((1, _HID, _g), jnp.float32),
            pltpu.VMEM((2 * _L,), jnp.float32),
            pltpu.SemaphoreType.DMA,
            pltpu.SemaphoreType.DMA,
            pltpu.SemaphoreType.DMA,
            pltpu.SemaphoreType.DMA,
            pltpu.SemaphoreType.DMA,
        ],
    )(_sc_body)
    return f(x2, w1t, w2)


def kernel(x, W1, W2):
    x2 = x.reshape(_B * _C, _HW)
    w1t = W1.transpose(0, 2, 1)
    out2 = _sc_call(x2, w1t, W2)
    return out2.reshape(_B, _C, _H, _W)


# speedup vs baseline: 10.4807x; 1.0379x over previous
import functools

import jax
import jax.numpy as jnp
from jax import lax
from jax.experimental import pallas as pl
from jax.experimental.pallas import tpu as pltpu
from jax.experimental.pallas import tpu_sc as plsc

_B, _C, _H, _W = 8, 768, 56, 56
_G = 4
_g = _C // _G
_HID = 12
_HW = _H * _W
_L = 16
_NV = _g // _L
_NS = _HW // _L
_CHUNK = 16
_NCHUNK = _g // _CHUNK


def _lane_total(scr, v):
    for k in (8, 4, 2, 1):
        scr[pl.ds(0, _L)] = v
        scr[pl.ds(_L, _L)] = v
        v = v + scr[pl.ds(k, _L)]
    return v


def _lane_splat(scr, tvec, v, j):
    onehot = jnp.where(tvec == j, 1.0, 0.0)
    return _lane_total(scr, v * onehot)


_UNROLL = 14


def _row_sum_vec(buf, j):
    zero = jnp.zeros((_L,), jnp.float32)

    def step(t, accs):
        accs = list(accs)
        base = t * _UNROLL * _L
        for u in range(_UNROLL):
            accs[u % 7] = accs[u % 7] + buf[j, pl.ds(base + u * _L, _L)]
        return tuple(accs)

    accs = lax.fori_loop(0, _NS // _UNROLL, step, (zero,) * 7)
    a = accs[0]
    for v in accs[1:]:
        a = a + v
    return a


def _sc_body(x_hbm, w1t_hbm, w2_hbm, out_hbm,
             buf0, buf1, idx0, idx1, pooled_v, att_v, w1_v, w2_v, rot_v,
             gsem0, gsem1, ssem0, ssem1, wsem):
    wid = lax.axis_index("s") * 2 + lax.axis_index("c")
    b = wid // _G
    i = wid % _G

    bufs = (buf0, buf1)
    idxs = (idx0, idx1)
    gsems = (gsem0, gsem1)
    ssems = (ssem0, ssem1)

    pltpu.async_copy(w1t_hbm.at[pl.ds(i, 1)], w1_v, wsem).wait()
    pltpu.async_copy(w2_hbm.at[pl.ds(i, 1)], w2_v, wsem).wait()

    row0 = b * _C + i
    tvec = lax.iota(jnp.int32, _L)

    def set_idx(s, c):
        idxs[s][...] = (row0 + 4 * _CHUNK * c) + 4 * tvec

    def start_gather(s, c):
        set_idx(s, c)
        pltpu.make_async_copy(x_hbm.at[idxs[s]], bufs[s], gsems[s]).start()

    def wait_gather(s):
        pltpu.make_async_copy(x_hbm.at[idxs[s]], bufs[s], gsems[s]).wait()

    start_gather(0, 0)
    for c in range(_NCHUNK):
        s = c % 2
        if c + 1 < _NCHUNK:
            start_gather(1 - s, c + 1)
        wait_gather(s)

        def red_row(j, sums_vec):
            sj = _lane_total(rot_v, _row_sum_vec(bufs[s], j))
            return jnp.where(tvec == j, sj, sums_vec)

        sums_vec = lax.fori_loop(0, _CHUNK, red_row,
                                 jnp.zeros((_L,), jnp.float32))
        pooled_v[pl.ds(c * _CHUNK, _CHUNK)] = sums_vec * (1.0 / _HW)

    hs = []
    for k in range(_HID):
        acc = jnp.zeros((_L,), jnp.float32)
        for m in range(_NV):
            acc = acc + (pooled_v[pl.ds(m * _L, _L)]
                         * w1_v[0, k, pl.ds(m * _L, _L)])
        hs.append(jnp.maximum(_lane_total(rot_v, acc), 0.0))
    for m in range(_NV):
        acc = jnp.zeros((_L,), jnp.float32)
        for k in range(_HID):
            acc = acc + hs[k] * w2_v[0, k, pl.ds(m * _L, _L)]
        att_v[pl.ds(m * _L, _L)] = 1.0 / (1.0 + jnp.exp(-acc))

    def start_scatter(s, c):
        pltpu.make_async_copy(bufs[s], out_hbm.at[idxs[s]], ssems[s]).start()

    def wait_scatter(s, c):
        pltpu.make_async_copy(bufs[s], out_hbm.at[idxs[s]], ssems[s]).wait()

    start_gather(0, 0)
    for c in range(_NCHUNK):
        s = c % 2
        wait_gather(s)
        if c + 1 < _NCHUNK:
            if c >= 1:
                wait_scatter(1 - s, c - 1)
            start_gather(1 - s, c + 1)

        av = att_v[pl.ds(c * _CHUNK, _CHUNK)]

        def scale_row(j, _):
            sj = _lane_splat(rot_v, tvec, av, j)

            def sstep(t, _2):
                base = t * _UNROLL * _L
                for u in range(_UNROLL):
                    sl = pl.ds(base + u * _L, _L)
                    bufs[s][j, sl] = bufs[s][j, sl] * sj
                return 0

            lax.fori_loop(0, _NS // _UNROLL, sstep, 0)
            return 0

        lax.fori_loop(0, _CHUNK, scale_row, 0)
        start_scatter(s, c)
    wait_scatter(0, _NCHUNK - 2)
    wait_scatter(1, _NCHUNK - 1)


@functools.partial(jax.jit, static_argnums=())
def _sc_call(x2, w1t, w2):
    mesh = plsc.VectorSubcoreMesh(core_axis_name="c", subcore_axis_name="s")
    f = functools.partial(
        pl.kernel,
        out_type=jax.ShapeDtypeStruct((_B * _C, _HW), jnp.float32),
        mesh=mesh,
        compiler_params=pltpu.CompilerParams(use_tc_tiling_on_sc=False),
        scratch_types=[
            pltpu.VMEM((_CHUNK, _HW), jnp.float32),
            pltpu.VMEM((_CHUNK, _HW), jnp.float32),
            pltpu.VMEM((_L,), jnp.int32),
            pltpu.VMEM((_L,), jnp.int32),
            pltpu.VMEM((_g,), jnp.float32),
            pltpu.VMEM((_g,), jnp.float32),
            pltpu.VMEM((1, _HID, _g), jnp.float32),
            pltpu.VMEM((1, _HID, _g), jnp.float32),
            pltpu.VMEM((2 * _L,), jnp.float32),
            pltpu.SemaphoreType.DMA,
            pltpu.SemaphoreType.DMA,
            pltpu.SemaphoreType.DMA,
            pltpu.SemaphoreType.DMA,
            pltpu.SemaphoreType.DMA,
        ],
    )(_sc_body)
    return f(x2, w1t, w2)


def kernel(x, W1, W2):
    x2 = x.reshape(_B * _C, _HW)
    w1t = W1.transpose(0, 2, 1)
    out2 = _sc_call(x2, w1t, W2)
    return out2.reshape(_B, _C, _H, _W)
